# baseline (speedup 1.0000x reference)
import jax
import jax.numpy as jnp
from jax.experimental import pallas as pl
from jax.experimental.pallas import tpu as pltpu

_N = 4096
_D = 512
_NCLS = 10
_K = 3
_BLK = 256
_NBLK = _N // _BLK
_BIG = 1e30


def _digamma(x):
    acc = jnp.zeros_like(x)
    for _ in range(6):
        acc = acc - 1.0 / x
        x = x + 1.0
    inv = 1.0 / x
    inv2 = inv * inv
    series = (jnp.log(x) - 0.5 * inv
              - inv2 * ((1.0 / 12.0) - inv2 * ((1.0 / 120.0) - inv2 * (1.0 / 252.0))))
    return series + acc


def _mi_block_kernel(xb_ref, xf_ref, ycol_ref, yrow_ref, out_ref, sqf_ref):
    step = pl.program_id(0)

    @pl.when(step == 0)
    def _prologue():
        xf = xf_ref[...]
        sqf_ref[...] = jnp.sum(xf * xf, axis=1)[None, :]

    xb = xb_ref[...]
    dotm2 = jax.lax.dot_general(xb * -2.0, xf_ref[...], (((1,), (1,)), ((), ())),
                                preferred_element_type=jnp.float32)
    sqb = jnp.sum(xb * xb, axis=1, keepdims=True)
    d2 = jnp.maximum((sqb + sqf_ref[...]) + dotm2, 0.0)

    yrow = yrow_ref[...]
    same = ycol_ref[...] == yrow
    w = jnp.where(same, d2, _BIG)

    v1 = jnp.min(w, axis=1, keepdims=True)
    cnt = jnp.sum(jnp.where(w == v1, 1.0, 0.0), axis=1, keepdims=True)
    anchor = v1
    cur = v1
    for _ in range(_K):
        nxt = jnp.min(jnp.where(w > cur, w, _BIG), axis=1, keepdims=True)
        c = jnp.sum(jnp.where(w == nxt, 1.0, 0.0), axis=1, keepdims=True)
        take = cnt < (_K + 1)
        anchor = jnp.where(take, nxt, anchor)
        cnt = cnt + jnp.where(take, c, 0.0)
        cur = nxt

    m = jnp.sum(jnp.where(d2 <= anchor, 1.0, 0.0), axis=1, keepdims=True) - 1.0
    dig_m = jnp.sum(_digamma(m + 1e-7))

    cls_sum = jnp.zeros((), jnp.float32)
    for c in range(_NCLS):
        n_c = jnp.sum(jnp.where(yrow == float(c), 1.0, 0.0))
        cls_sum = cls_sum + n_c * _digamma(n_c)
    dig_s = jnp.where(step == 0, cls_sum, 0.0)

    lane = jax.lax.broadcasted_iota(jnp.int32, (1, 1, 2), 2)
    out_ref[...] = jnp.where(lane == 0, dig_m, dig_s)


def kernel(X, y):
    yf = y.astype(jnp.float32)
    ycol = yf.reshape(_N, 1)
    yrow = yf.reshape(1, _N)

    partials = pl.pallas_call(
        _mi_block_kernel,
        grid=(_NBLK,),
        in_specs=[
            pl.BlockSpec((_BLK, _D), lambda i: (i, 0)),
            pl.BlockSpec((_N, _D), lambda i: (0, 0)),
            pl.BlockSpec((_BLK, 1), lambda i: (i, 0)),
            pl.BlockSpec((1, _N), lambda i: (0, 0)),
        ],
        out_specs=pl.BlockSpec((1, 1, 2), lambda i: (i, 0, 0)),
        out_shape=jax.ShapeDtypeStruct((_NBLK, 1, 2), jnp.float32),
        scratch_shapes=[pltpu.VMEM((1, _N), jnp.float32)],
        compiler_params=pltpu.CompilerParams(
            dimension_semantics=("arbitrary",)),
    )(X, X, ycol, yrow)

    sums = jnp.sum(partials, axis=(0, 1))
    avg_m = sums[0] / _N
    avg_nx = sums[1] / _N
    dig_n = jax.scipy.special.digamma(jnp.float32(_N))
    dig_k = jax.scipy.special.digamma(jnp.float32(_K))
    mi = (dig_n - avg_nx + dig_k - avg_m) / jnp.log(jnp.float32(2.0))
    return jax.nn.relu(mi)


# speedup vs baseline: 1.4828x; 1.4828x over previous
import jax
import jax.numpy as jnp
from jax.experimental import pallas as pl
from jax.experimental.pallas import tpu as pltpu

_N = 4096
_D = 512
_NCLS = 10
_K = 3
_BLK = 256
_NBLK = _N // _BLK
_BIG = 1e30


def _digamma(x):
    acc = jnp.zeros_like(x)
    for _ in range(6):
        acc = acc - 1.0 / x
        x = x + 1.0
    inv = 1.0 / x
    inv2 = inv * inv
    series = (jnp.log(x) - 0.5 * inv
              - inv2 * ((1.0 / 12.0) - inv2 * ((1.0 / 120.0) - inv2 * (1.0 / 252.0))))
    return series + acc


def _mi_block_kernel(xb_ref, xf_ref, ycol_ref, yrow_ref, out_ref, sqf_ref,
                     cls_ref):
    step = pl.program_id(0)

    @pl.when(step == 0)
    def _prologue():
        xf = xf_ref[...]
        sqf_ref[...] = jnp.sum(xf * xf, axis=1)[None, :]
        yr = yrow_ref[...]
        cls_sum = jnp.zeros((), jnp.float32)
        for c in range(_NCLS):
            n_c = jnp.sum(jnp.where(yr == float(c), 1.0, 0.0))
            cls_sum = cls_sum + n_c * _digamma(n_c)
        lane0 = jax.lax.broadcasted_iota(jnp.int32, (1, 128), 1) == 0
        cls_ref[...] = jnp.where(lane0, cls_sum, 0.0)

    xb = xb_ref[...]
    dotm2 = jax.lax.dot_general(xb * -2.0, xf_ref[...], (((1,), (1,)), ((), ())),
                                preferred_element_type=jnp.float32)
    sqb = jnp.sum(xb * xb, axis=1, keepdims=True)
    d2 = (sqb + sqf_ref[...]) + dotm2

    yrow = yrow_ref[...]
    same = ycol_ref[...] == yrow
    w = jnp.where(same, d2, _BIG)

    m1 = jnp.full((_BLK, 128), _BIG, jnp.float32)
    m2 = jnp.full((_BLK, 128), _BIG, jnp.float32)
    m3 = jnp.full((_BLK, 128), _BIG, jnp.float32)
    m4 = jnp.full((_BLK, 128), _BIG, jnp.float32)
    for t in range(_N // 128):
        e = w[:, 128 * t:128 * (t + 1)]
        m4 = jnp.minimum(m4, jnp.maximum(m3, e))
        m3 = jnp.minimum(m3, jnp.maximum(m2, e))
        m2 = jnp.minimum(m2, jnp.maximum(m1, e))
        m1 = jnp.minimum(m1, e)
    u = jnp.concatenate([m1, m2, m3, m4], axis=1)

    v1 = jnp.min(u, axis=1, keepdims=True)
    cnt = jnp.sum(jnp.where(u == v1, 1.0, 0.0), axis=1, keepdims=True)
    anchor = v1
    cur = v1
    for _ in range(_K):
        nxt = jnp.min(jnp.where(u > cur, u, _BIG), axis=1, keepdims=True)
        c = jnp.sum(jnp.where(u == nxt, 1.0, 0.0), axis=1, keepdims=True)
        take = cnt < (_K + 1)
        anchor = jnp.where(take, nxt, anchor)
        cnt = cnt + jnp.where(take, c, 0.0)
        cur = nxt

    m = jnp.sum(jnp.where(d2 <= anchor, 1.0, 0.0), axis=1, keepdims=True) - 1.0
    dig_m = jnp.sum(_digamma(m + 1e-7))

    dig_s = jnp.where(step == 0, cls_ref[0, 0], 0.0)

    lane = jax.lax.broadcasted_iota(jnp.int32, (1, 1, 2), 2)
    out_ref[...] = jnp.where(lane == 0, dig_m, dig_s)


def kernel(X, y):
    yf = y.astype(jnp.float32)
    ycol = yf.reshape(_N, 1)
    yrow = yf.reshape(1, _N)

    partials = pl.pallas_call(
        _mi_block_kernel,
        grid=(_NBLK,),
        in_specs=[
            pl.BlockSpec((_BLK, _D), lambda i: (i, 0)),
            pl.BlockSpec((_N, _D), lambda i: (0, 0)),
            pl.BlockSpec((_BLK, 1), lambda i: (i, 0)),
            pl.BlockSpec((1, _N), lambda i: (0, 0)),
        ],
        out_specs=pl.BlockSpec((1, 1, 2), lambda i: (i, 0, 0)),
        out_shape=jax.ShapeDtypeStruct((_NBLK, 1, 2), jnp.float32),
        scratch_shapes=[pltpu.VMEM((1, _N), jnp.float32),
                        pltpu.VMEM((1, 128), jnp.float32)],
        compiler_params=pltpu.CompilerParams(
            dimension_semantics=("arbitrary",)),
    )(X, X, ycol, yrow)

    sums = jnp.sum(partials, axis=(0, 1))
    avg_m = sums[0] / _N
    avg_nx = sums[1] / _N
    dig_n = jax.scipy.special.digamma(jnp.float32(_N))
    dig_k = jax.scipy.special.digamma(jnp.float32(_K))
    mi = (dig_n - avg_nx + dig_k - avg_m) / jnp.log(jnp.float32(2.0))
    return jax.nn.relu(mi)


# speedup vs baseline: 1.5453x; 1.0422x over previous
import jax
import jax.numpy as jnp
from jax.experimental import pallas as pl
from jax.experimental.pallas import tpu as pltpu

_N = 4096
_D = 512
_NCLS = 10
_K = 3
_BLK = 256
_NBLK = _N // _BLK
_BIG = 1e30


def _digamma(x):
    acc = jnp.zeros_like(x)
    for _ in range(6):
        acc = acc - 1.0 / x
        x = x + 1.0
    inv = 1.0 / x
    inv2 = inv * inv
    series = (jnp.log(x) - 0.5 * inv
              - inv2 * ((1.0 / 12.0) - inv2 * ((1.0 / 120.0) - inv2 * (1.0 / 252.0))))
    return series + acc


def _mi_block_kernel(xb_ref, xf_ref, ycol_ref, yrow_ref, out_ref, sqf_ref,
                     cls_ref):
    step = pl.program_id(0)

    @pl.when(step == 0)
    def _prologue():
        xf = xf_ref[...]
        sqf_ref[...] = jnp.sum(xf * xf, axis=1)[None, :]
        yr = yrow_ref[...]
        cls_sum = jnp.zeros((), jnp.float32)
        for c in range(_NCLS):
            n_c = jnp.sum(jnp.where(yr == float(c), 1.0, 0.0))
            cls_sum = cls_sum + n_c * _digamma(n_c)
        lane0 = jax.lax.broadcasted_iota(jnp.int32, (1, 128), 1) == 0
        cls_ref[...] = jnp.where(lane0, cls_sum, 0.0)

    xb = xb_ref[...]
    dotm2 = jax.lax.dot_general(xb * -2.0, xf_ref[...], (((1,), (1,)), ((), ())),
                                preferred_element_type=jnp.float32)
    d2 = sqf_ref[...] + dotm2

    yrow = yrow_ref[...]
    same = ycol_ref[...] == yrow
    w = jnp.where(same, d2, _BIG)

    m1 = jnp.full((_BLK, 128), _BIG, jnp.float32)
    m2 = jnp.full((_BLK, 128), _BIG, jnp.float32)
    m3 = jnp.full((_BLK, 128), _BIG, jnp.float32)
    m4 = jnp.full((_BLK, 128), _BIG, jnp.float32)
    for t in range(_N // 128):
        e = w[:, 128 * t:128 * (t + 1)]
        m4 = jnp.minimum(m4, jnp.maximum(m3, e))
        m3 = jnp.minimum(m3, jnp.maximum(m2, e))
        m2 = jnp.minimum(m2, jnp.maximum(m1, e))
        m1 = jnp.minimum(m1, e)
    u = jnp.concatenate([m1, m2, m3, m4], axis=1)

    v1 = jnp.min(u, axis=1, keepdims=True)
    cnt = jnp.sum(jnp.where(u == v1, 1.0, 0.0), axis=1, keepdims=True)
    anchor = v1
    cur = v1
    for _ in range(_K):
        nxt = jnp.min(jnp.where(u > cur, u, _BIG), axis=1, keepdims=True)
        c = jnp.sum(jnp.where(u == nxt, 1.0, 0.0), axis=1, keepdims=True)
        take = cnt < (_K + 1)
        anchor = jnp.where(take, nxt, anchor)
        cnt = cnt + jnp.where(take, c, 0.0)
        cur = nxt

    m = jnp.sum(jnp.where(d2 <= anchor, 1.0, 0.0), axis=1, keepdims=True) - 1.0
    dig_m = jnp.sum(_digamma(m + 1e-7))

    dig_s = jnp.where(step == 0, cls_ref[0, 0], 0.0)

    lane = jax.lax.broadcasted_iota(jnp.int32, (1, 1, 2), 2)
    out_ref[...] = jnp.where(lane == 0, dig_m, dig_s)


def kernel(X, y):
    yf = y.astype(jnp.float32)
    ycol = yf.reshape(_N, 1)
    yrow = yf.reshape(1, _N)

    partials = pl.pallas_call(
        _mi_block_kernel,
        grid=(_NBLK,),
        in_specs=[
            pl.BlockSpec((_BLK, _D), lambda i: (i, 0)),
            pl.BlockSpec((_N, _D), lambda i: (0, 0)),
            pl.BlockSpec((_BLK, 1), lambda i: (i, 0)),
            pl.BlockSpec((1, _N), lambda i: (0, 0)),
        ],
        out_specs=pl.BlockSpec((1, 1, 2), lambda i: (i, 0, 0)),
        out_shape=jax.ShapeDtypeStruct((_NBLK, 1, 2), jnp.float32),
        scratch_shapes=[pltpu.VMEM((1, _N), jnp.float32),
                        pltpu.VMEM((1, 128), jnp.float32)],
        compiler_params=pltpu.CompilerParams(
            dimension_semantics=("arbitrary",)),
    )(X, X, ycol, yrow)

    sums = jnp.sum(partials, axis=(0, 1))
    avg_m = sums[0] / _N
    avg_nx = sums[1] / _N
    dig_n = jax.scipy.special.digamma(jnp.float32(_N))
    dig_k = jax.scipy.special.digamma(jnp.float32(_K))
    mi = (dig_n - avg_nx + dig_k - avg_m) / jnp.log(jnp.float32(2.0))
    return jax.nn.relu(mi)


# speedup vs baseline: 1.6194x; 1.0479x over previous
import jax
import jax.numpy as jnp
from jax.experimental import pallas as pl
from jax.experimental.pallas import tpu as pltpu

_N = 4096
_D = 512
_NCLS = 10
_K = 3
_BLK = 256
_NBLK = _N // _BLK
_BIG = 1e30


def _digamma(x):
    acc = jnp.zeros_like(x)
    for _ in range(6):
        acc = acc - 1.0 / x
        x = x + 1.0
    inv = 1.0 / x
    inv2 = inv * inv
    series = (jnp.log(x) - 0.5 * inv
              - inv2 * ((1.0 / 12.0) - inv2 * ((1.0 / 120.0) - inv2 * (1.0 / 252.0))))
    return series + acc


_DIG_N = 8.31764409143979
_DIG_K = 0.9227843350984671
_INV_LN2 = 1.4426950408889634


def _mi_block_kernel(xb_ref, xf_ref, ycol_ref, yrow_ref, out_ref, sqf_ref,
                     cls_ref, acc_ref):
    step = pl.program_id(0)

    @pl.when(step == 0)
    def _prologue():
        acc_ref[...] = jnp.zeros((1, 128), jnp.float32)
        xf = xf_ref[...]
        sqf_ref[...] = jnp.sum(xf * xf, axis=1)[None, :]
        yr = yrow_ref[...]
        cls_sum = jnp.zeros((), jnp.float32)
        for c in range(_NCLS):
            n_c = jnp.sum(jnp.where(yr == float(c), 1.0, 0.0))
            cls_sum = cls_sum + n_c * _digamma(n_c)
        lane0 = jax.lax.broadcasted_iota(jnp.int32, (1, 128), 1) == 0
        cls_ref[...] = jnp.where(lane0, cls_sum, 0.0)

    xb = xb_ref[...]
    dotm2 = jax.lax.dot_general(xb * -2.0, xf_ref[...], (((1,), (1,)), ((), ())),
                                preferred_element_type=jnp.float32)
    d2 = sqf_ref[...] + dotm2

    yrow = yrow_ref[...]
    same = ycol_ref[...] == yrow
    w = jnp.where(same, d2, _BIG)

    m1 = jnp.full((_BLK, 128), _BIG, jnp.float32)
    m2 = jnp.full((_BLK, 128), _BIG, jnp.float32)
    m3 = jnp.full((_BLK, 128), _BIG, jnp.float32)
    m4 = jnp.full((_BLK, 128), _BIG, jnp.float32)
    for t in range(_N // 128):
        e = w[:, 128 * t:128 * (t + 1)]
        m4 = jnp.minimum(m4, jnp.maximum(m3, e))
        m3 = jnp.minimum(m3, jnp.maximum(m2, e))
        m2 = jnp.minimum(m2, jnp.maximum(m1, e))
        m1 = jnp.minimum(m1, e)
    u = jnp.concatenate([m1, m2, m3, m4], axis=1)

    v1 = jnp.min(u, axis=1, keepdims=True)
    cnt = jnp.sum(jnp.where(u == v1, 1.0, 0.0), axis=1, keepdims=True)
    anchor = v1
    cur = v1
    for _ in range(_K):
        nxt = jnp.min(jnp.where(u > cur, u, _BIG), axis=1, keepdims=True)
        c = jnp.sum(jnp.where(u == nxt, 1.0, 0.0), axis=1, keepdims=True)
        take = cnt < (_K + 1)
        anchor = jnp.where(take, nxt, anchor)
        cnt = cnt + jnp.where(take, c, 0.0)
        cur = nxt

    m = jnp.sum(jnp.where(d2 <= anchor, 1.0, 0.0), axis=1, keepdims=True) - 1.0
    dig_m = jnp.sum(_digamma(m + 1e-7))

    acc = acc_ref[...] + dig_m
    acc_ref[...] = acc

    @pl.when(step == _NBLK - 1)
    def _epilogue():
        avg_m = acc[0, 0] * (1.0 / _N)
        avg_nx = cls_ref[0, 0] * (1.0 / _N)
        mi = (_DIG_N - avg_nx + _DIG_K - avg_m) * _INV_LN2
        out_ref[...] = jnp.full((1, 128), jnp.maximum(mi, 0.0), jnp.float32)


def kernel(X, y):
    yf = y.astype(jnp.float32)
    ycol = yf.reshape(_N, 1)
    yrow = yf.reshape(1, _N)

    out = pl.pallas_call(
        _mi_block_kernel,
        grid=(_NBLK,),
        in_specs=[
            pl.BlockSpec((_BLK, _D), lambda i: (i, 0)),
            pl.BlockSpec((_N, _D), lambda i: (0, 0)),
            pl.BlockSpec((_BLK, 1), lambda i: (i, 0)),
            pl.BlockSpec((1, _N), lambda i: (0, 0)),
        ],
        out_specs=pl.BlockSpec((1, 128), lambda i: (0, 0)),
        out_shape=jax.ShapeDtypeStruct((1, 128), jnp.float32),
        scratch_shapes=[pltpu.VMEM((1, _N), jnp.float32),
                        pltpu.VMEM((1, 128), jnp.float32),
                        pltpu.VMEM((1, 128), jnp.float32)],
        compiler_params=pltpu.CompilerParams(
            dimension_semantics=("arbitrary",)),
    )(X, X, ycol, yrow)

    return out[0, 0]
